# baseline (speedup 1.0000x reference)
import functools

import jax
import jax.numpy as jnp
from jax import lax
from jax.experimental import pallas as pl
from jax.experimental.pallas import tpu as pltpu
from jax.experimental.pallas import tpu_sc as plsc

B = 16384
D = 64
NC = 2
NS = 16
NW = NC * NS
L = 16
RING = 4

F_TC = 4096
B_SC = B - F_TC
BPW = B_SC // NW
NG = BPW // L
TCB = 128
NSTEP = F_TC // TCB

BPWB = B // NW
CHUNK = 128
NCHUNK = BPWB // CHUNK


def _dot_body(user_hbm, item_hbm, ut_hbm, it_hbm,
              out_hbm,
              uidx_v, iidx_v, ubufs_v, ibufs_v, out_v, sem):
    wid = lax.axis_index("s") * NC + lax.axis_index("c")

    pltpu.sync_copy(user_hbm.at[wid], uidx_v.at[pl.ds(0, BPW)])
    pltpu.sync_copy(item_hbm.at[wid], iidx_v.at[pl.ds(0, BPW)])

    lane = lax.iota(jnp.int32, 16)
    dnums = lax.GatherDimensionNumbers(
        offset_dims=(), collapsed_slice_dims=(0,), start_index_map=(0,))

    def shuffle(x, idx):
        return lax.gather(x, idx[:, None], dnums, (1,),
                          mode=lax.GatherScatterMode.PROMISE_IN_BOUNDS)

    def fire(uvec, ivec, r, slot):
        ublk = pl.multiple_of((uvec[r] >> 7) * 128, 128)
        iblk = pl.multiple_of((ivec[r] >> 7) * 128, 128)
        pltpu.async_copy(ut_hbm.at[:, pl.ds(ublk, 128)],
                         ubufs_v.at[slot], sem)
        pltpu.async_copy(it_hbm.at[:, pl.ds(iblk, 128)],
                         ibufs_v.at[slot], sem)

    def drain(slot):
        pltpu.make_async_copy(ut_hbm.at[:, pl.ds(0, 128)],
                              ubufs_v.at[slot], sem).wait()
        pltpu.make_async_copy(it_hbm.at[:, pl.ds(0, 128)],
                              ibufs_v.at[slot], sem).wait()

    def process(uvec, ivec, r, slot, acc):
        drain(slot)
        cu = uvec[r] & 127
        ci = ivec[r] & 127
        cu_al = cu & ~15
        ci_al = ci & ~15
        ulane = jnp.broadcast_to(cu & 15, (L,))
        ilane = jnp.broadcast_to(ci & 15, (L,))

        def dstep(k, p):
            d = k * 8
            for dd in range(8):
                u16 = ubufs_v[slot, d + dd, pl.ds(cu_al, 16)]
                i16 = ibufs_v[slot, d + dd, pl.ds(ci_al, 16)]
                p = p + shuffle(u16, ulane) * shuffle(i16, ilane)
            return p

        p = lax.fori_loop(0, D // 8, dstep, jnp.zeros((L,), jnp.float32))
        return jnp.where(lane == r, p, acc)

    uvec0 = uidx_v[pl.ds(0, L)]
    ivec0 = iidx_v[pl.ds(0, L)]
    for r in range(RING):
        fire(uvec0, ivec0, r, r)

    def group(g, carry):
        uvec, ivec = carry
        unext = uidx_v[pl.ds((g + 1) * L, L)]
        inext = iidx_v[pl.ds((g + 1) * L, L)]
        acc = jnp.zeros((L,), jnp.float32)
        for r in range(L):
            acc = process(uvec, ivec, r, r % RING, acc)
            if r < L - RING:
                fire(uvec, ivec, r + RING, (r + RING) % RING)
            else:
                @pl.when(g < NG - 1)
                def _():
                    fire(unext, inext, r + RING - L, (r + RING) % RING)
        out_v[pl.ds(g * L, L)] = acc
        return (unext, inext)

    lax.fori_loop(0, NG, group, (uvec0, ivec0))

    pltpu.sync_copy(out_v, out_hbm.at[pl.ds(wid * BPW, BPW)])


def _tc_body(uidx_ref, iidx_ref, ut_hbm, it_hbm, out_ref,
             ublk, iblk, sem0, sem1):
    g = pl.program_id(0)
    lane128 = lax.broadcasted_iota(jnp.int32, (1, TCB), 1)
    sems = (sem0, sem1)

    def fire(step, parity):
        sem = sems[parity]

        def body(jj, carry):
            u = uidx_ref[step * TCB + jj]
            i = iidx_ref[step * TCB + jj]
            ub = pl.multiple_of((u >> 7) * 128, 128)
            ib = pl.multiple_of((i >> 7) * 128, 128)
            pltpu.make_async_copy(ut_hbm.at[:, pl.ds(ub, 128)],
                                  ublk.at[parity, jj], sem).start()
            pltpu.make_async_copy(it_hbm.at[:, pl.ds(ib, 128)],
                                  iblk.at[parity, jj], sem).start()
            return carry

        lax.fori_loop(0, TCB, body, 0)

    def drain(parity):
        sem = sems[parity]

        def body(jj, carry):
            pltpu.make_async_copy(ut_hbm.at[:, pl.ds(0, 128)],
                                  ublk.at[parity, jj], sem).wait()
            pltpu.make_async_copy(it_hbm.at[:, pl.ds(0, 128)],
                                  iblk.at[parity, jj], sem).wait()
            return carry

        lax.fori_loop(0, TCB, body, 0)

    @pl.when(g == 0)
    def _():
        fire(0, 0)

    for p in range(2):
        @pl.when(((g + 1) < NSTEP) & ((g + 1) % 2 == p))
        def _():
            fire(g + 1, p)

    parity = g % 2
    for p in range(2):
        @pl.when(parity == p)
        def _():
            drain(p)

    def compute(jj, acc):
        u = uidx_ref[g * TCB + jj]
        i = iidx_ref[g * TCB + jj]
        cu = u & 127
        ci = i & 127
        bu = ublk[parity, jj]
        bi = iblk[parity, jj]
        ucol = jnp.sum(bu * jnp.where(lane128 == cu, 1.0, 0.0),
                       axis=1, keepdims=True)
        icol = jnp.sum(bi * jnp.where(lane128 == ci, 1.0, 0.0),
                       axis=1, keepdims=True)
        s = jnp.sum(ucol * icol)
        return jnp.where(lane128 == jj, s, acc)

    acc = lax.fori_loop(0, TCB, compute, jnp.zeros((1, TCB), jnp.float32))
    out_ref[0, pl.ds(0, 1), :] = acc


def _bias_body(user_hbm, item_hbm, dot_hbm, bu_hbm, bi_hbm, gb_hbm,
               out_hbm,
               uidx_v, iidx_v, ubias_v, ibias_v, dot_v, gb_v, sem):
    wid = lax.axis_index("s") * NC + lax.axis_index("c")

    pltpu.sync_copy(user_hbm.at[wid], uidx_v)
    pltpu.sync_copy(item_hbm.at[wid], iidx_v)
    pltpu.sync_copy(dot_hbm.at[pl.ds(wid * BPWB, BPWB)], dot_v)
    pltpu.sync_copy(gb_hbm, gb_v)

    copies = []
    for j in range(NCHUNK):
        sl = pl.ds(j * CHUNK, CHUNK)
        copies.append(pltpu.async_copy(bu_hbm.at[uidx_v.at[sl]],
                                       ubias_v.at[sl], sem))
        copies.append(pltpu.async_copy(bi_hbm.at[iidx_v.at[sl]],
                                       ibias_v.at[sl], sem))
    for cp in copies:
        cp.wait()

    gbv = gb_v[...]

    def finish(g, carry):
        sl = pl.ds(g * L, L)
        res = dot_v[sl] + gbv + ubias_v[sl] + ibias_v[sl]
        dot_v[sl] = jnp.minimum(jnp.maximum(res, 1.0), 5.0)
        return carry

    lax.fori_loop(0, BPWB // L, finish, 0)

    pltpu.sync_copy(dot_v, out_hbm.at[pl.ds(wid * BPWB, BPWB)])


@jax.jit
def _svd_score(user_r, item_r, user_tc, item_tc, ut_t, it_t,
               bias_user_flat, bias_item_flat, gb16):
    mesh = plsc.VectorSubcoreMesh(core_axis_name="c", subcore_axis_name="s")
    dot_k = functools.partial(
        pl.kernel,
        out_type=jax.ShapeDtypeStruct((B_SC,), jnp.float32),
        mesh=mesh,
        scratch_types=[
            pltpu.VMEM((BPW + L,), jnp.int32),
            pltpu.VMEM((BPW + L,), jnp.int32),
            pltpu.VMEM((RING, D, 128), jnp.float32),
            pltpu.VMEM((RING, D, 128), jnp.float32),
            pltpu.VMEM((BPW,), jnp.float32),
            pltpu.SemaphoreType.DMA,
        ],
        compiler_params=pltpu.CompilerParams(use_tc_tiling_on_sc=True),
    )(_dot_body)
    dot_sc = dot_k(user_r, item_r, ut_t, it_t)

    dot_tc = pl.pallas_call(
        _tc_body,
        grid=(NSTEP,),
        in_specs=[
            pl.BlockSpec(memory_space=pltpu.SMEM),
            pl.BlockSpec(memory_space=pltpu.SMEM),
            pl.BlockSpec(memory_space=pltpu.MemorySpace.HBM),
            pl.BlockSpec(memory_space=pltpu.MemorySpace.HBM),
        ],
        out_specs=pl.BlockSpec((1, 8, TCB), lambda g: (g, 0, 0)),
        out_shape=jax.ShapeDtypeStruct((NSTEP, 8, TCB), jnp.float32),
        scratch_shapes=[
            pltpu.VMEM((2, TCB, D, 128), jnp.float32),
            pltpu.VMEM((2, TCB, D, 128), jnp.float32),
            pltpu.SemaphoreType.DMA,
            pltpu.SemaphoreType.DMA,
        ],
        compiler_params=pltpu.CompilerParams(
            dimension_semantics=("arbitrary",)),
    )(user_tc, item_tc, ut_t, it_t)

    dot = jnp.concatenate([dot_sc, dot_tc[:, 0, :].reshape(F_TC)])

    bias_k = functools.partial(
        pl.kernel,
        out_type=jax.ShapeDtypeStruct((B,), jnp.float32),
        mesh=mesh,
        scratch_types=[
            pltpu.VMEM((BPWB,), jnp.int32),
            pltpu.VMEM((BPWB,), jnp.int32),
            pltpu.VMEM((BPWB,), jnp.float32),
            pltpu.VMEM((BPWB,), jnp.float32),
            pltpu.VMEM((BPWB,), jnp.float32),
            pltpu.VMEM((16,), jnp.float32),
            pltpu.SemaphoreType.DMA,
        ],
        compiler_params=pltpu.CompilerParams(use_tc_tiling_on_sc=False),
    )(_bias_body)
    user_b = jnp.concatenate([user_r.reshape(B_SC), user_tc]).reshape(NW, BPWB)
    item_b = jnp.concatenate([item_r.reshape(B_SC), item_tc]).reshape(NW, BPWB)
    return bias_k(user_b, item_b, dot, bias_user_flat, bias_item_flat, gb16)


def kernel(user, item, user_table, item_table, bias_user_table,
           bias_item_table, global_bias):
    gb16 = jnp.broadcast_to(
        jnp.asarray(global_bias, jnp.float32).reshape(1), (16,))
    out = _svd_score(user[:B_SC].reshape(NW, BPW), item[:B_SC].reshape(NW, BPW),
                     user[B_SC:], item[B_SC:],
                     user_table.T, item_table.T,
                     bias_user_table.reshape(-1), bias_item_table.reshape(-1),
                     gb16)
    return out.reshape(1, B)


# speedup vs baseline: 2.7962x; 2.7962x over previous
import functools

import jax
import jax.numpy as jnp
from jax import lax
from jax.experimental import pallas as pl
from jax.experimental.pallas import tpu as pltpu
from jax.experimental.pallas import tpu_sc as plsc

B = 16384
D = 64
NC = 2
NS = 16
NW = NC * NS
L = 16
RING = 4

BPW = B // NW
NG = BPW // L

BPWB = B // NW
CHUNK = 128
NCHUNK = BPWB // CHUNK


def _dot_body(user_hbm, item_hbm, ut_hbm, it_hbm,
              out_hbm,
              uidx_v, iidx_v, ubufs_v, ibufs_v, out_v, sem):
    wid = lax.axis_index("s") * NC + lax.axis_index("c")

    pltpu.sync_copy(user_hbm.at[wid], uidx_v.at[pl.ds(0, BPW)])
    pltpu.sync_copy(item_hbm.at[wid], iidx_v.at[pl.ds(0, BPW)])

    lane = lax.iota(jnp.int32, 16)
    dnums = lax.GatherDimensionNumbers(
        offset_dims=(), collapsed_slice_dims=(0,), start_index_map=(0,))

    def shuffle(x, idx):
        return lax.gather(x, idx[:, None], dnums, (1,),
                          mode=lax.GatherScatterMode.PROMISE_IN_BOUNDS)

    def fire(uvec, ivec, r, slot):
        ublk = pl.multiple_of((uvec[r] >> 7) * 128, 128)
        iblk = pl.multiple_of((ivec[r] >> 7) * 128, 128)
        pltpu.async_copy(ut_hbm.at[:, pl.ds(ublk, 128)],
                         ubufs_v.at[slot], sem)
        pltpu.async_copy(it_hbm.at[:, pl.ds(iblk, 128)],
                         ibufs_v.at[slot], sem)

    def drain(slot):
        pltpu.make_async_copy(ut_hbm.at[:, pl.ds(0, 128)],
                              ubufs_v.at[slot], sem).wait()
        pltpu.make_async_copy(it_hbm.at[:, pl.ds(0, 128)],
                              ibufs_v.at[slot], sem).wait()

    def process(uvec, ivec, r, slot, acc):
        drain(slot)
        cu = uvec[r] & 127
        ci = ivec[r] & 127
        cu_al = cu & ~15
        ci_al = ci & ~15
        ulane = jnp.broadcast_to(cu & 15, (L,))
        ilane = jnp.broadcast_to(ci & 15, (L,))

        def dstep(k, p):
            d = k * 8
            for dd in range(8):
                u16 = ubufs_v[slot, d + dd, pl.ds(cu_al, 16)]
                i16 = ibufs_v[slot, d + dd, pl.ds(ci_al, 16)]
                p = p + shuffle(u16, ulane) * shuffle(i16, ilane)
            return p

        p = lax.fori_loop(0, D // 8, dstep, jnp.zeros((L,), jnp.float32))
        return jnp.where(lane == r, p, acc)

    uvec0 = uidx_v[pl.ds(0, L)]
    ivec0 = iidx_v[pl.ds(0, L)]
    for r in range(RING):
        fire(uvec0, ivec0, r, r)

    def group(g, carry):
        uvec, ivec = carry
        unext = uidx_v[pl.ds((g + 1) * L, L)]
        inext = iidx_v[pl.ds((g + 1) * L, L)]
        acc = jnp.zeros((L,), jnp.float32)
        for r in range(L):
            acc = process(uvec, ivec, r, r % RING, acc)
            if r < L - RING:
                fire(uvec, ivec, r + RING, (r + RING) % RING)
            else:
                @pl.when(g < NG - 1)
                def _():
                    fire(unext, inext, r + RING - L, (r + RING) % RING)
        out_v[pl.ds(g * L, L)] = acc
        return (unext, inext)

    lax.fori_loop(0, NG, group, (uvec0, ivec0))

    pltpu.sync_copy(out_v, out_hbm.at[pl.ds(wid * BPW, BPW)])


def _bias_body(user_hbm, item_hbm, dot_hbm, bu_hbm, bi_hbm, gb_hbm,
               out_hbm,
               uidx_v, iidx_v, ubias_v, ibias_v, dot_v, gb_v, sem):
    wid = lax.axis_index("s") * NC + lax.axis_index("c")

    pltpu.sync_copy(user_hbm.at[wid], uidx_v)
    pltpu.sync_copy(item_hbm.at[wid], iidx_v)
    pltpu.sync_copy(dot_hbm.at[pl.ds(wid * BPWB, BPWB)], dot_v)
    pltpu.sync_copy(gb_hbm, gb_v)

    copies = []
    for j in range(NCHUNK):
        sl = pl.ds(j * CHUNK, CHUNK)
        copies.append(pltpu.async_copy(bu_hbm.at[uidx_v.at[sl]],
                                       ubias_v.at[sl], sem))
        copies.append(pltpu.async_copy(bi_hbm.at[iidx_v.at[sl]],
                                       ibias_v.at[sl], sem))
    for cp in copies:
        cp.wait()

    gbv = gb_v[...]

    def finish(g, carry):
        sl = pl.ds(g * L, L)
        res = dot_v[sl] + gbv + ubias_v[sl] + ibias_v[sl]
        dot_v[sl] = jnp.minimum(jnp.maximum(res, 1.0), 5.0)
        return carry

    lax.fori_loop(0, BPWB // L, finish, 0)

    pltpu.sync_copy(dot_v, out_hbm.at[pl.ds(wid * BPWB, BPWB)])


@jax.jit
def _svd_score(user_r, item_r, ut_t, it_t,
               bias_user_flat, bias_item_flat, gb16):
    mesh = plsc.VectorSubcoreMesh(core_axis_name="c", subcore_axis_name="s")
    dot_k = functools.partial(
        pl.kernel,
        out_type=jax.ShapeDtypeStruct((B,), jnp.float32),
        mesh=mesh,
        scratch_types=[
            pltpu.VMEM((BPW + L,), jnp.int32),
            pltpu.VMEM((BPW + L,), jnp.int32),
            pltpu.VMEM((RING, D, 128), jnp.float32),
            pltpu.VMEM((RING, D, 128), jnp.float32),
            pltpu.VMEM((BPW,), jnp.float32),
            pltpu.SemaphoreType.DMA,
        ],
        compiler_params=pltpu.CompilerParams(use_tc_tiling_on_sc=True),
    )(_dot_body)
    dot = dot_k(user_r, item_r, ut_t, it_t)

    bias_k = functools.partial(
        pl.kernel,
        out_type=jax.ShapeDtypeStruct((B,), jnp.float32),
        mesh=mesh,
        scratch_types=[
            pltpu.VMEM((BPWB,), jnp.int32),
            pltpu.VMEM((BPWB,), jnp.int32),
            pltpu.VMEM((BPWB,), jnp.float32),
            pltpu.VMEM((BPWB,), jnp.float32),
            pltpu.VMEM((BPWB,), jnp.float32),
            pltpu.VMEM((16,), jnp.float32),
            pltpu.SemaphoreType.DMA,
        ],
        compiler_params=pltpu.CompilerParams(use_tc_tiling_on_sc=False),
    )(_bias_body)
    return bias_k(user_r, item_r, dot, bias_user_flat, bias_item_flat, gb16)


def kernel(user, item, user_table, item_table, bias_user_table,
           bias_item_table, global_bias):
    gb16 = jnp.broadcast_to(
        jnp.asarray(global_bias, jnp.float32).reshape(1), (16,))
    out = _svd_score(user.reshape(NW, BPW), item.reshape(NW, BPW),
                     user_table.T, item_table.T,
                     bias_user_table.reshape(-1), bias_item_table.reshape(-1),
                     gb16)
    return out.reshape(1, B)
